# baseline (speedup 1.0000x reference)
import functools

import jax
import jax.numpy as jnp
from jax import lax
from jax.experimental import pallas as pl
from jax.experimental.pallas import tpu as pltpu
from jax.experimental.pallas import tpu_sc as plsc

N = 10000
E = 320000
NPD = 10240
NPT = 640
NC, NS = 2, 16
NW = NC * NS
NT = 624
NREM = N - NS * NT
WIN = 128
NWIN = 80
EPW = NWIN * WIN
E_PAD = NW * EPW
CWC = 40
NCHUNK = NWIN // CWC
CH = 128
R = 1000

_mesh = plsc.VectorSubcoreMesh(core_axis_name="c", subcore_axis_name="s")
_sc_params = pltpu.CompilerParams(needs_layout_passes=False)


def _zero_vmem(buf, rows):
    z = jnp.zeros((16,), jnp.float32)

    def body(r, carry):
        for j in range(CH // 16):
            buf[r, pl.ds(16 * j, 16)] = z
        return carry

    lax.fori_loop(0, rows, body, 0)


def _zero_1d(buf, chunks):
    z = jnp.zeros((16,), jnp.float32)

    def body(i, carry):
        buf[pl.ds(16 * i, 16)] = z
        return carry

    lax.fori_loop(0, chunks, body, 0)


@functools.partial(
    pl.kernel,
    out_type=jax.ShapeDtypeStruct((NC, NPD), jnp.float32),
    mesh=_mesh,
    compiler_params=_sc_params,
    scratch_types=[
        pltpu.VMEM((NWIN, WIN), jnp.int32),
        pltpu.VMEM((NWIN, WIN), jnp.float32),
        pltpu.VMEM((NPT,), jnp.float32),
        pltpu.VMEM_SHARED((NPD,), jnp.float32),
    ],
)
def _sc_deg(col_hbm, we_hbm, deg_hbm, col_v, w_v, dbuf, deg_sh):
    c = lax.axis_index("c")
    s = lax.axis_index("s")
    wid = c * NS + s
    base = s * NPT

    _zero_1d(dbuf, NPT // 16)
    pltpu.sync_copy(dbuf, deg_sh.at[pl.ds(base, NPT)])
    pltpu.sync_copy(col_hbm.at[wid], col_v)
    pltpu.sync_copy(we_hbm.at[wid], w_v)
    plsc.subcore_barrier()

    def dscat(i, carry):
        pltpu.sync_copy(w_v.at[i], deg_sh.at[col_v.at[i]], add=True)
        return carry

    lax.fori_loop(0, NWIN, dscat, 0)
    plsc.subcore_barrier()
    pltpu.sync_copy(deg_sh.at[pl.ds(base, NPT)],
                    deg_hbm.at[c, pl.ds(base, NPT)])


@functools.partial(
    pl.kernel,
    out_type=jax.ShapeDtypeStruct((NW, NWIN, WIN), jnp.float32),
    mesh=_mesh,
    compiler_params=_sc_params,
    scratch_types=[
        pltpu.VMEM((NWIN, WIN), jnp.int32),
        pltpu.VMEM((NWIN, WIN), jnp.int32),
        pltpu.VMEM((NWIN, WIN), jnp.float32),
        pltpu.VMEM((8, WIN), jnp.float32),
        pltpu.VMEM((8, WIN), jnp.float32),
        pltpu.SemaphoreType.DMA,
    ],
)
def _sc_norm(row_hbm, col_hbm, we_hbm, dis_hbm, norm_hbm,
             row_v, col_v, nv, dr, dc, sem):
    c = lax.axis_index("c")
    s = lax.axis_index("s")
    wid = c * NS + s

    pltpu.sync_copy(row_hbm.at[wid], row_v)
    pltpu.sync_copy(col_hbm.at[wid], col_v)
    pltpu.sync_copy(we_hbm.at[wid], nv)

    for sub in range(NWIN // 8):
        for b in range(8):
            w = 8 * sub + b
            pltpu.make_async_copy(
                dis_hbm.at[row_v.at[w]], dr.at[b], sem).start()
            pltpu.make_async_copy(
                dis_hbm.at[col_v.at[w]], dc.at[b], sem).start()
        for b in range(8):
            w = 8 * sub + b
            pltpu.make_async_copy(
                dis_hbm.at[row_v.at[w]], dr.at[b], sem).wait()
            pltpu.make_async_copy(
                dis_hbm.at[col_v.at[w]], dc.at[b], sem).wait()
        for b in range(8):
            w = 8 * sub + b
            for k in range(WIN // 16):
                sl = pl.ds(16 * k, 16)
                nv[w, sl] = dr[b, sl] * nv[w, sl] * dc[b, sl]

    pltpu.sync_copy(nv, norm_hbm.at[wid])


def _scale(gb, nrmc, w):
    w16 = jnp.full((16,), w, jnp.int32)

    def rows(k, carry):
        for rsub in range(4):
            r = 4 * k + rsub
            nb = plsc.load_gather(nrmc, [w16, jnp.full((16,), r, jnp.int32)])
            for j in range(CH // 16):
                sl = pl.ds(16 * j, 16)
                gb[r, sl] = gb[r, sl] * nb
        return carry

    lax.fori_loop(0, WIN // 4, rows, 0)


@functools.partial(
    pl.kernel,
    out_type=jax.ShapeDtypeStruct((NC, N, CH), jnp.float32),
    mesh=_mesh,
    compiler_params=_sc_params,
    scratch_types=[
        pltpu.VMEM((CWC, WIN), jnp.int32),
        pltpu.VMEM((CWC, WIN), jnp.int32),
        pltpu.VMEM((CWC, WIN), jnp.float32),
        pltpu.VMEM((WIN, CH), jnp.float32),
        pltpu.VMEM((WIN, CH), jnp.float32),
        pltpu.VMEM((8, CH), jnp.float32),
        pltpu.VMEM_SHARED((N, CH), jnp.float32),
        pltpu.SemaphoreType.DMA,
        pltpu.SemaphoreType.DMA,
        pltpu.SemaphoreType.DMA,
        pltpu.SemaphoreType.DMA,
    ],
)
def _sc_agg(row_hbm, col_hbm, norm_hbm, h_hbm, g_hbm,
            rowc, colc, nrmc, gbA, gbB, zbuf, acc_sh, gsA, gsB, ssA, ssB):
    c = lax.axis_index("c")
    s = lax.axis_index("s")
    wid = c * NS + s
    base = s * NT
    gb = (gbA, gbB)
    gs = (gsA, gsB)
    ss = (ssA, ssB)

    def gst(q, w):
        pltpu.make_async_copy(h_hbm.at[rowc.at[w]], gb[q], gs[q]).start()

    def gwt(q, w):
        pltpu.make_async_copy(h_hbm.at[rowc.at[w]], gb[q], gs[q]).wait()

    def sst(q, w):
        pltpu.make_async_copy(
            gb[q], acc_sh.at[colc.at[w]], ss[q]).start(add=True)

    def swt(q, w):
        pltpu.make_async_copy(gb[q], acc_sh.at[colc.at[w]], ss[q]).wait()

    _zero_vmem(zbuf, 8)

    def zslab(i, carry):
        pltpu.sync_copy(zbuf, acc_sh.at[pl.ds(base + 8 * i, 8)])
        return carry

    lax.fori_loop(0, NT // 8, zslab, 0)

    @pl.when(s == NS - 1)
    def _():
        pltpu.sync_copy(zbuf.at[pl.ds(0, 8)], acc_sh.at[pl.ds(NS * NT, 8)])
        pltpu.sync_copy(zbuf.at[pl.ds(0, 8)],
                        acc_sh.at[pl.ds(NS * NT + 8, 8)])

    plsc.subcore_barrier()

    for ci in range(NCHUNK):
        csl = pl.ds(CWC * ci, CWC)
        pltpu.sync_copy(row_hbm.at[wid, csl], rowc)
        pltpu.sync_copy(col_hbm.at[wid, csl], colc)
        pltpu.sync_copy(norm_hbm.at[wid, csl], nrmc)
        gst(0, 0)

        def pair(p, carry):
            w0 = 2 * p
            w1 = w0 + 1

            @pl.when(p > 0)
            def _():
                swt(1, w1 - 2)

            gst(1, w1)
            gwt(0, w0)
            _scale(gbA, nrmc, w0)
            sst(0, w0)
            gwt(1, w1)
            _scale(gbB, nrmc, w1)
            sst(1, w1)

            @pl.when(p < CWC // 2 - 1)
            def _():
                swt(0, w0)
                gst(0, w0 + 2)

            return carry

        lax.fori_loop(0, CWC // 2, pair, 0)
        swt(0, CWC - 2)
        swt(1, CWC - 1)

    plsc.subcore_barrier()
    pltpu.sync_copy(acc_sh.at[pl.ds(base, NT)], g_hbm.at[c, pl.ds(base, NT)])

    @pl.when(s == NS - 1)
    def _():
        pltpu.sync_copy(acc_sh.at[pl.ds(NS * NT, NREM)],
                        g_hbm.at[c, pl.ds(NS * NT, NREM)])


def _prep_body(degp_ref, x_ref, w_ref, dis_ref, inv_ref, h_ref):
    deg = 1.0 + degp_ref[0] + degp_ref[1]
    dis_ref[...] = lax.rsqrt(deg)
    inv_ref[...] = 1.0 / deg
    h_ref[...] = jnp.dot(x_ref[...], w_ref[...],
                         preferred_element_type=jnp.float32)


def _tc_prep(degp3, x, w):
    return pl.pallas_call(
        _prep_body,
        grid=(N // R,),
        in_specs=[pl.BlockSpec((NC, R, 1), lambda i: (0, i, 0)),
                  pl.BlockSpec((R, CH), lambda i: (i, 0)),
                  pl.BlockSpec((CH, CH), lambda i: (0, 0))],
        out_specs=[pl.BlockSpec((R, 1), lambda i: (i, 0)),
                   pl.BlockSpec((R, 1), lambda i: (i, 0)),
                   pl.BlockSpec((R, CH), lambda i: (i, 0))],
        out_shape=[jax.ShapeDtypeStruct((N, 1), jnp.float32),
                   jax.ShapeDtypeStruct((N, 1), jnp.float32),
                   jax.ShapeDtypeStruct((N, CH), jnp.float32)],
    )(degp3, x, w)


def _comb_body(g_ref, h_ref, inv_ref, b_ref, w_ref, o_ref):
    x = jnp.maximum(g_ref[0] + g_ref[1] + inv_ref[...] * h_ref[...]
                    + b_ref[...], 0.0)
    o_ref[...] = jnp.dot(x, w_ref[...], preferred_element_type=jnp.float32)


def _tc_comb_mm(g, h, inv2, b2d, w):
    return pl.pallas_call(
        _comb_body,
        grid=(N // R,),
        in_specs=[pl.BlockSpec((NC, R, CH), lambda i: (0, i, 0)),
                  pl.BlockSpec((R, CH), lambda i: (i, 0)),
                  pl.BlockSpec((R, 1), lambda i: (i, 0)),
                  pl.BlockSpec((1, CH), lambda i: (0, 0)),
                  pl.BlockSpec((CH, CH), lambda i: (0, 0))],
        out_specs=pl.BlockSpec((R, CH), lambda i: (i, 0)),
        out_shape=jax.ShapeDtypeStruct((N, CH), jnp.float32),
    )(g, h, inv2, b2d, w)


def _combx_body(g_ref, h_ref, inv_ref, b_ref, o_ref):
    o_ref[...] = jnp.maximum(g_ref[0] + g_ref[1]
                             + inv_ref[...] * h_ref[...] + b_ref[...], 0.0)


def _tc_comb_x(g, h, inv2, b2d):
    return pl.pallas_call(
        _combx_body,
        grid=(N // R,),
        in_specs=[pl.BlockSpec((NC, R, CH), lambda i: (0, i, 0)),
                  pl.BlockSpec((R, CH), lambda i: (i, 0)),
                  pl.BlockSpec((R, 1), lambda i: (i, 0)),
                  pl.BlockSpec((1, CH), lambda i: (0, 0))],
        out_specs=pl.BlockSpec((R, CH), lambda i: (i, 0)),
        out_shape=jax.ShapeDtypeStruct((N, CH), jnp.float32),
    )(g, h, inv2, b2d)


def _final_body(g_ref, x_ref, inv_ref, wmu_ref, bmu_ref, wls_ref, bls_ref,
                mu_ref, ls_ref):
    y = g_ref[0] + g_ref[1] + inv_ref[...] * x_ref[...]
    mu_ref[...] = jnp.dot(y, wmu_ref[...],
                          preferred_element_type=jnp.float32) + bmu_ref[...]
    ls_ref[...] = jnp.dot(y, wls_ref[...],
                          preferred_element_type=jnp.float32) + bls_ref[...]


def _tc_final(g, x3, inv2, wmu, bmu2, wls, bls2):
    oc = wmu.shape[1]
    return pl.pallas_call(
        _final_body,
        grid=(N // R,),
        in_specs=[pl.BlockSpec((NC, R, CH), lambda i: (0, i, 0)),
                  pl.BlockSpec((R, CH), lambda i: (i, 0)),
                  pl.BlockSpec((R, 1), lambda i: (i, 0)),
                  pl.BlockSpec((CH, oc), lambda i: (0, 0)),
                  pl.BlockSpec((1, oc), lambda i: (0, 0)),
                  pl.BlockSpec((CH, oc), lambda i: (0, 0)),
                  pl.BlockSpec((1, oc), lambda i: (0, 0))],
        out_specs=[pl.BlockSpec((R, oc), lambda i: (i, 0)),
                   pl.BlockSpec((R, oc), lambda i: (i, 0))],
        out_shape=[jax.ShapeDtypeStruct((N, oc), jnp.float32),
                   jax.ShapeDtypeStruct((N, oc), jnp.float32)],
    )(g, x3, inv2, wmu, bmu2, wls, bls2)


def kernel(X, edge_index, edge_weight, W1, b1, W2, b2, W3, b3,
           Wmu, bmu, Wls, bls):
    pad = E_PAD - E
    rowp = jnp.pad(edge_index[0], (0, pad)).reshape(NW, NWIN, WIN)
    colp = jnp.pad(edge_index[1], (0, pad)).reshape(NW, NWIN, WIN)
    wp = jnp.pad(edge_weight, (0, pad)).reshape(NW, NWIN, WIN)
    b1r, b2r, b3r = b1.reshape(1, CH), b2.reshape(1, CH), b3.reshape(1, CH)
    bmur, blsr = bmu.reshape(1, -1), bls.reshape(1, -1)

    degp = _sc_deg(colp, wp)
    dis2, inv2, h1 = _tc_prep(degp[:, :N].reshape(NC, N, 1), X, W1)
    normp = _sc_norm(rowp, colp, wp, dis2.reshape(N))
    g1 = _sc_agg(rowp, colp, normp, h1)
    h2 = _tc_comb_mm(g1, h1, inv2, b1r, W2)
    g2 = _sc_agg(rowp, colp, normp, h2)
    h3 = _tc_comb_mm(g2, h2, inv2, b2r, W3)
    g3 = _sc_agg(rowp, colp, normp, h3)
    x3 = _tc_comb_x(g3, h3, inv2, b3r)
    g4 = _sc_agg(rowp, colp, normp, x3)
    mu, ls = _tc_final(g4, x3, inv2, Wmu, bmur, Wls, blsr)
    return mu, ls
